# baseline (speedup 1.0000x reference)
import functools

import jax
import jax.numpy as jnp
from jax.experimental import pallas as pl
from jax.experimental.pallas import tpu as pltpu
from jax.experimental.pallas import tpu_sc as plsc


def _dist_body(z_ref, cb_ref, idx_ref, loss_ref):
    b = pl.program_id(0)
    zb = z_ref[0]
    cb = cb_ref[...]

    cnorm = jnp.sum(cb * cb, axis=1)
    scores = jax.lax.dot_general(
        cb, zb, (((1,), (0,)), ((), ())),
        preferred_element_type=jnp.float32)
    d = cnorm[:, None] - 2.0 * scores

    dmin = jnp.min(d, axis=0)
    idx = jnp.argmin(d, axis=0).astype(jnp.int32)

    xnorm = jnp.sum(zb * zb, axis=0)
    loss_part = jnp.sum(dmin + xnorm)

    idx_ref[0, 0] = idx
    loss_blk = jnp.reshape(loss_part, (1, 1))

    @pl.when(b == 0)
    def _init():
        loss_ref[...] = loss_blk

    @pl.when(b > 0)
    def _acc():
        loss_ref[...] += loss_blk


def _dist_argmin(z3, codebook):
    B, C, N = z3.shape
    K = codebook.shape[0]
    return pl.pallas_call(
        _dist_body,
        grid=(B,),
        in_specs=[
            pl.BlockSpec((1, C, N), lambda b: (b, 0, 0)),
            pl.BlockSpec((K, C), lambda b: (0, 0)),
        ],
        out_specs=[
            pl.BlockSpec((1, 1, N), lambda b: (b, 0, 0)),
            pl.BlockSpec((1, 1), lambda b: (0, 0)),
        ],
        out_shape=[
            jax.ShapeDtypeStruct((B, 1, N), jnp.int32),
            jax.ShapeDtypeStruct((1, 1), jnp.float32),
        ],
    )(z3, codebook)


_SC_CORES = 2
_SC_SUBCORES = 16
_SC_WORKERS = _SC_CORES * _SC_SUBCORES


def _sc_gather(codebook, idx_flat):
    T = idx_flat.shape[0]
    C = codebook.shape[1]
    per_w = T // _SC_WORKERS
    CH = 128
    n_ch = per_w // CH
    mesh = plsc.VectorSubcoreMesh(core_axis_name="c", subcore_axis_name="s")

    @functools.partial(
        pl.kernel, mesh=mesh,
        out_type=jax.ShapeDtypeStruct((T, C), jnp.float32),
        scratch_types=[
            pltpu.VMEM((CH,), jnp.int32),
            pltpu.VMEM((CH, C), jnp.float32),
            pltpu.SemaphoreType.DMA,
        ],
    )
    def k(table_hbm, idx_hbm, out_hbm, idx_v, rows_v, sem):
        wid = jax.lax.axis_index("s") * _SC_CORES + jax.lax.axis_index("c")
        base = wid * per_w
        for ch in range(n_ch):
            off = base + ch * CH
            pltpu.sync_copy(idx_hbm.at[pl.ds(off, CH)], idx_v)
            pltpu.async_copy(table_hbm.at[idx_v], rows_v, sem).wait()
            pltpu.sync_copy(rows_v, out_hbm.at[pl.ds(off, CH)])

    return k(codebook, idx_flat)


@jax.jit
def _vq(z, codebook):
    b, c, h, w = z.shape
    z3 = z.reshape(b, c, h * w)
    idx, loss = _dist_argmin(z3, codebook)
    zq_flat = _sc_gather(codebook, idx.reshape(-1))
    z_q_out = zq_flat.reshape(b, h, w, c).transpose(0, 3, 1, 2)
    codebook_loss = loss[0, 0] * 1.25 / (b * c * h * w)
    indices_out = idx.reshape(b, 1, h, w)
    return (z_q_out, codebook_loss, indices_out)


def kernel(z, embedding_weight):
    return _vq(z, embedding_weight)
